# baseline (speedup 1.0000x reference)
import functools

import jax
import jax.numpy as jnp
from jax import lax
from jax.experimental import pallas as pl
from jax.experimental.pallas import tpu as pltpu
from jax.experimental.pallas import tpu_sc as plsc

VOCAB = 1000000
EMBED_DIM = 32
B = 16384
L = 50

NC = 2
NS = 16
NW = NC * NS

TOTAL = B * L
B_PER_W = TOTAL // NW
G = 10
CH = G * 128
NCHUNK = B_PER_W // CH


def _emb_body(idx_hbm, table_hbm, out_hbm, idx_v, rows_v,
              gsem0, gsem1, ssem0, ssem1):
  c = lax.axis_index("c")
  s = lax.axis_index("s")
  wid = s * NC + c
  gsems = (gsem0, gsem1)
  ssems = (ssem0, ssem1)

  def fire(i, b):
    pltpu.sync_copy(idx_hbm.at[pl.ds((wid * NCHUNK + i) * G, G), :], idx_v.at[b])
    for j in range(G):
      pltpu.async_copy(
          table_hbm.at[idx_v.at[b, j]],
          rows_v.at[b, pl.ds(j * 128, 128)],
          gsems[b],
      )

  def drain(b):
    for j in range(G):
      pltpu.make_async_copy(
          table_hbm.at[idx_v.at[b, j]],
          rows_v.at[b, pl.ds(j * 128, 128)],
          gsems[b],
      ).wait()

  def store(i, b):
    pltpu.async_copy(
        rows_v.at[b], out_hbm.at[pl.ds((wid * NCHUNK + i) * CH, CH), :],
        ssems[b])

  def wait_store(i, b):
    pltpu.make_async_copy(
        rows_v.at[b], out_hbm.at[pl.ds((wid * NCHUNK + i) * CH, CH), :],
        ssems[b]).wait()

  fire(0, 0)
  fire(1, 1)

  def outer(i):
    drain(0)
    store(i, 0)

    @pl.when(i + 2 < NCHUNK)
    def _():
      wait_store(i, 0)
      fire(i + 2, 0)

    drain(1)
    store(i + 1, 1)

    @pl.when(i + 3 < NCHUNK)
    def _():
      wait_store(i + 1, 1)
      fire(i + 3, 1)

  pl.loop(0, NCHUNK, step=2)(outer)
  wait_store(NCHUNK - 2, 0)
  wait_store(NCHUNK - 1, 1)


@jax.jit
def _embedding_sc(batch, weight):
  idx = batch.reshape(TOTAL // 128, 128)
  mesh = plsc.VectorSubcoreMesh(core_axis_name="c", subcore_axis_name="s")
  out = pl.kernel(
      _emb_body,
      out_type=jax.ShapeDtypeStruct((TOTAL, EMBED_DIM), jnp.float32),
      mesh=mesh,
      scratch_types=[
          pltpu.VMEM((2, G, 128), jnp.int32),
          pltpu.VMEM((2, CH, EMBED_DIM), jnp.float32),
          pltpu.SemaphoreType.DMA,
          pltpu.SemaphoreType.DMA,
          pltpu.SemaphoreType.DMA,
          pltpu.SemaphoreType.DMA,
      ],
      compiler_params=pltpu.CompilerParams(use_tc_tiling_on_sc=False),
  )(idx, weight)
  return out.reshape(B, L, EMBED_DIM)


def kernel(batch, weight):
  return _embedding_sc(batch, weight)


# speedup vs baseline: 1.7143x; 1.7143x over previous
import jax
import jax.numpy as jnp
from jax import lax
from jax.experimental import pallas as pl
from jax.experimental.pallas import tpu as pltpu
from jax.experimental.pallas import tpu_sc as plsc

VOCAB = 1000000
EMBED_DIM = 32
B = 16384
L = 50

NC = 2
NS = 16
NW = NC * NS

BSLAB = B // NW
G = BSLAB // 128


def _emb_body(idx_hbm, table_hbm, out_hbm, idx_v, rows_v,
              gsem0, gsem1, ssem0, ssem1):
  c = lax.axis_index("c")
  s = lax.axis_index("s")
  wid = s * NC + c
  gsems = (gsem0, gsem1)
  ssems = (ssem0, ssem1)

  def fire(l, b):
    pltpu.sync_copy(idx_hbm.at[l, pl.ds(wid * G, G), :], idx_v.at[b])
    for j in range(G):
      pltpu.async_copy(
          table_hbm.at[idx_v.at[b, j]],
          rows_v.at[b, pl.ds(j * 128, 128)],
          gsems[b],
      )

  def drain(b):
    for j in range(G):
      pltpu.make_async_copy(
          table_hbm.at[idx_v.at[b, j]],
          rows_v.at[b, pl.ds(j * 128, 128)],
          gsems[b],
      ).wait()

  def store(l, b):
    pltpu.async_copy(
        rows_v.at[b], out_hbm.at[l, pl.ds(wid * BSLAB, BSLAB), :], ssems[b])

  def wait_store(l, b):
    pltpu.make_async_copy(
        rows_v.at[b], out_hbm.at[l, pl.ds(wid * BSLAB, BSLAB), :],
        ssems[b]).wait()

  fire(0, 0)
  fire(1, 1)

  def outer(l):
    drain(0)
    store(l, 0)

    @pl.when(l + 2 < L)
    def _():
      wait_store(l, 0)
      fire(l + 2, 0)

    drain(1)
    store(l + 1, 1)

    @pl.when(l + 3 < L)
    def _():
      wait_store(l + 1, 1)
      fire(l + 3, 1)

  pl.loop(0, L, step=2)(outer)
  wait_store(L - 2, 0)
  wait_store(L - 1, 1)


@jax.jit
def _embedding_sc(batch, weight):
  idx = batch.T.reshape(L, B // 128, 128)
  mesh = plsc.VectorSubcoreMesh(core_axis_name="c", subcore_axis_name="s")
  out = pl.kernel(
      _emb_body,
      out_type=jax.ShapeDtypeStruct((L, B, EMBED_DIM), jnp.float32),
      mesh=mesh,
      scratch_types=[
          pltpu.VMEM((2, G, 128), jnp.int32),
          pltpu.VMEM((2, BSLAB, EMBED_DIM), jnp.float32),
          pltpu.SemaphoreType.DMA,
          pltpu.SemaphoreType.DMA,
          pltpu.SemaphoreType.DMA,
          pltpu.SemaphoreType.DMA,
      ],
      compiler_params=pltpu.CompilerParams(use_tc_tiling_on_sc=False),
  )(idx, weight)
  return out.transpose(1, 0, 2)


def kernel(batch, weight):
  return _embedding_sc(batch, weight)
